# baseline (speedup 1.0000x reference)
import functools

import jax
import jax.numpy as jnp
from jax import lax
from jax.experimental import pallas as pl
from jax.experimental.pallas import tpu as pltpu
from jax.experimental.pallas import tpu_sc as plsc

NUM_CLASSES = 1000000
EMB_CHANNEL = 64
BATCH = 16384

NC = 2
NS = 16
NW = NC * NS
B_PER_W = BATCH // NW
CHUNK = 128
NCHUNK = B_PER_W // CHUNK


@functools.lru_cache(maxsize=1)
def _build_gather():
    mesh = plsc.VectorSubcoreMesh(core_axis_name="c", subcore_axis_name="s")

    @functools.partial(
        pl.kernel,
        mesh=mesh,
        out_type=jax.ShapeDtypeStruct((BATCH, EMB_CHANNEL), jnp.float32),
        scratch_types=[
            pltpu.VMEM((NCHUNK, CHUNK), jnp.int32),
            pltpu.VMEM((B_PER_W, EMB_CHANNEL), jnp.float32),
            pltpu.SemaphoreType.DMA,
        ],
        compiler_params=pltpu.CompilerParams(use_tc_tiling_on_sc=False),
    )
    def emb_gather(table_hbm, idx_hbm, out_hbm, idx_v, rows_v, sem):
        wid = lax.axis_index("s") * NC + lax.axis_index("c")
        pltpu.sync_copy(idx_hbm.at[pl.ds(wid * NCHUNK, NCHUNK)], idx_v)
        copies = []
        for j in range(NCHUNK):
            copies.append(
                pltpu.async_copy(
                    table_hbm.at[idx_v.at[j]],
                    rows_v.at[pl.ds(j * CHUNK, CHUNK)],
                    sem,
                )
            )
        for c in copies:
            c.wait()
        pltpu.sync_copy(rows_v, out_hbm.at[pl.ds(wid * B_PER_W, B_PER_W)])

    return emb_gather


def kernel(class_labels, label_emb_weight):
    idx = class_labels.astype(jnp.int32).reshape(NW * NCHUNK, CHUNK)
    return _build_gather()(label_emb_weight, idx)


# speedup vs baseline: 1.2841x; 1.2841x over previous
import functools

import jax
import jax.numpy as jnp
from jax import lax
from jax.experimental import pallas as pl
from jax.experimental.pallas import tpu as pltpu
from jax.experimental.pallas import tpu_sc as plsc

NUM_CLASSES = 1000000
EMB_CHANNEL = 64
BATCH = 16384

NC = 2
NS = 16
NW = NC * NS
B_PER_W = BATCH // NW
ROWS_PER_TILE = 8
NTILE = NUM_CLASSES // ROWS_PER_TILE


@functools.lru_cache(maxsize=1)
def _build_gather():
    mesh = plsc.VectorSubcoreMesh(core_axis_name="c", subcore_axis_name="s")

    @functools.partial(
        pl.kernel,
        mesh=mesh,
        out_type=jax.ShapeDtypeStruct((BATCH, EMB_CHANNEL), jnp.float32),
        scratch_types=[
            pltpu.VMEM_SHARED((NS, B_PER_W), jnp.int32),
            pltpu.SMEM((B_PER_W,), jnp.int32),
            pltpu.SemaphoreType.DMA,
        ],
    )
    def emb_gather(table_hbm, idx_hbm, out_hbm, idx_sh, idx_s, sem):
        cid = lax.axis_index("c")
        sid = lax.axis_index("s")
        wid = sid * NC + cid
        base = wid * B_PER_W
        pltpu.sync_copy(idx_hbm.at[pl.ds(base, B_PER_W)], idx_sh.at[sid])
        pltpu.sync_copy(idx_sh.at[sid], idx_s)

        def body(j, _):
            i = idx_s[j]
            pltpu.async_copy(
                table_hbm.at[i >> 3, i & 7], out_hbm.at[base + j], sem
            )
            return _

        lax.fori_loop(0, B_PER_W, body, None)
        pltpu.make_async_copy(
            out_hbm.at[pl.ds(base, B_PER_W)],
            out_hbm.at[pl.ds(base, B_PER_W)],
            sem,
        ).wait()

    return emb_gather


def kernel(class_labels, label_emb_weight):
    idx = class_labels.astype(jnp.int32)
    table3 = label_emb_weight.reshape(NTILE, ROWS_PER_TILE, EMB_CHANNEL)
    return _build_gather()(table3, idx)


# speedup vs baseline: 1.7240x; 1.3426x over previous
import functools

import jax
import jax.numpy as jnp
from jax import lax
from jax.experimental import pallas as pl
from jax.experimental.pallas import tpu as pltpu
from jax.experimental.pallas import tpu_sc as plsc

NUM_CLASSES = 1000000
EMB_CHANNEL = 64
BATCH = 16384

NC = 2
NS = 16
NW = NC * NS
B_PER_W = BATCH // NW


@functools.lru_cache(maxsize=1)
def _build_gather():
    mesh = plsc.VectorSubcoreMesh(core_axis_name="c", subcore_axis_name="s")

    @functools.partial(
        pl.kernel,
        mesh=mesh,
        out_type=jax.ShapeDtypeStruct((BATCH, EMB_CHANNEL), jnp.float32),
        scratch_types=[
            pltpu.VMEM_SHARED((NS, B_PER_W), jnp.int32),
            pltpu.SMEM((B_PER_W,), jnp.int32),
            pltpu.VMEM((B_PER_W, EMB_CHANNEL), jnp.float32),
            pltpu.SemaphoreType.DMA,
        ],
    )
    def emb_gather(table_hbm, idx_hbm, out_hbm, idx_sh, idx_s, rows_v, sem):
        cid = lax.axis_index("c")
        sid = lax.axis_index("s")
        wid = sid * NC + cid
        base = wid * B_PER_W
        pltpu.sync_copy(idx_hbm.at[pl.ds(base, B_PER_W)], idx_sh.at[sid])
        pltpu.sync_copy(idx_sh.at[sid], idx_s)

        def body(j, _):
            pltpu.async_copy(table_hbm.at[idx_s[j]], rows_v.at[j], sem)
            return _

        lax.fori_loop(0, B_PER_W, body, None)
        pltpu.make_async_copy(
            out_hbm.at[pl.ds(base, B_PER_W)], rows_v, sem
        ).wait()
        pltpu.sync_copy(rows_v, out_hbm.at[pl.ds(base, B_PER_W)])

    return emb_gather


def kernel(class_labels, label_emb_weight):
    idx = class_labels.astype(jnp.int32)
    return _build_gather()(label_emb_weight, idx)
